# baseline (speedup 1.0000x reference)
import functools

import jax
import jax.numpy as jnp
from jax import lax
from jax.experimental import pallas as pl
from jax.experimental.pallas import tpu as pltpu
from jax.experimental.pallas import tpu_sc as plsc

_EMB = 32
_NUM_CORES = 2
_NUM_SUBCORES = 16
_NW = _NUM_CORES * _NUM_SUBCORES
_BLK = 1024
_B = 16384
_F = 26
_GPF = _B // _BLK
_NBLK = _F * _GPF // _NW


_NTILE = 7812
_TPW = _NTILE // _NW


def _make_transpose():
    mesh = plsc.VectorSubcoreMesh(core_axis_name="c", subcore_axis_name="s")

    @functools.partial(
        pl.kernel,
        mesh=mesh,
        out_type=jax.ShapeDtypeStruct((250016, 128), jnp.float32),
        scratch_types=[
            [pltpu.VMEM((32, 128), jnp.float32) for _ in range(2)],
            [pltpu.VMEM((32, 129), jnp.float32) for _ in range(2)],
            pltpu.VMEM((16, 128), jnp.float32),
            [pltpu.SemaphoreType.DMA for _ in range(2)],
            [pltpu.SemaphoreType.DMA for _ in range(2)],
        ],
        compiler_params=pltpu.CompilerParams(
            use_tc_tiling_on_sc=True, needs_layout_passes=False
        ),
    )
    def trans_kernel(tabt, tail, out, ibufs, tbufs, tailbuf, isems, osems):
        wid = lax.axis_index("s") * _NUM_CORES + lax.axis_index("c")
        base = wid * _TPW
        iota = lax.iota(jnp.int32, 16)

        def issue_in(tt, b):
            pltpu.async_copy(
                tabt.at[:, pl.ds(pl.multiple_of(tt * 128, 128), 128)],
                ibufs[b],
                isems[b],
            )

        def wait_in(tt, b):
            pltpu.make_async_copy(
                tabt.at[:, pl.ds(pl.multiple_of(tt * 128, 128), 128)],
                ibufs[b],
                isems[b],
            ).wait()

        def issue_out(tt, b):
            pltpu.async_copy(
                tbufs[b].at[:, pl.ds(0, 128)],
                out.at[pl.ds(pl.multiple_of(tt * 32, 32), 32), :],
                osems[b],
            )

        def wait_out(tt, b):
            pltpu.make_async_copy(
                tbufs[b].at[:, pl.ds(0, 128)],
                out.at[pl.ds(pl.multiple_of(tt * 32, 32), 32), :],
                osems[b],
            ).wait()

        def transpose(b):
            @plsc.parallel_loop(0, 32, step=1, unroll=4)
            def _(d):
                for c0 in range(0, 128, 16):
                    cv = c0 + iota
                    rowv = cv >> 2
                    colb = (cv & 3) * 32
                    vals = ibufs[b][d, pl.ds(c0, 16)]
                    plsc.store_scatter(tbufs[b], [rowv, colb + d], vals)

        issue_in(base, 0)
        issue_in(base + 1, 1)

        def body(kk, carry):
            for b in range(2):
                k = kk * 2 + b
                wait_in(base + k, b)

                @pl.when(kk >= 1)
                def _():
                    wait_out(base + k - 2, b)

                transpose(b)

                @pl.when(kk < _TPW // 2 - 1)
                def _():
                    issue_in(base + k + 2, b)

                issue_out(base + k, b)
            return carry

        lax.fori_loop(0, _TPW // 2, body, 0)
        for b in range(2):
            wait_out(base + _TPW - 2 + b, b)

        @pl.when(wid < _NTILE - _TPW * _NW)
        def _():
            tt = _TPW * _NW + wid
            pltpu.async_copy(
                tabt.at[:, pl.ds(pl.multiple_of(tt * 128, 128), 128)],
                ibufs[0],
                isems[0],
            ).wait()
            transpose(0)
            pltpu.async_copy(
                tbufs[0].at[:, pl.ds(0, 128)],
                out.at[pl.ds(pl.multiple_of(tt * 32, 32), 32), :],
                osems[0],
            ).wait()

        @pl.when(wid == 4)
        def _():
            pltpu.sync_copy(tail, tailbuf)
            pltpu.sync_copy(tailbuf, out.at[pl.ds(_NTILE * 32, 16), :])

    return trans_kernel


def _make_gather():
    n = _B * _F
    per_w = n // _NW
    mesh = plsc.VectorSubcoreMesh(core_axis_name="c", subcore_axis_name="s")

    @functools.partial(
        pl.kernel,
        mesh=mesh,
        out_type=jax.ShapeDtypeStruct((_F * _EMB * _B // 128, 128), jnp.float32),
        scratch_types=[
            pltpu.VMEM((per_w,), jnp.int32),
            [pltpu.VMEM((_BLK, _EMB), jnp.float32) for _ in range(2)],
            pltpu.VMEM((_BLK * _EMB // 128, 129), jnp.float32),
            [pltpu.SemaphoreType.DMA for _ in range(2)],
            [pltpu.SemaphoreType.DMA for _ in range(2)],
        ],
        compiler_params=pltpu.CompilerParams(
            use_tc_tiling_on_sc=False, needs_layout_passes=False
        ),
    )
    def emb_kernel(idx_hbm, table_hbm, out_hbm, idx_v, gbufs, tbuf, gsems, tsems):
        wid = lax.axis_index("s") * _NUM_CORES + lax.axis_index("c")
        base = wid * per_w
        iota = lax.iota(jnp.int32, 16)
        rvec0 = (iota // 8) * 64 + iota % 8

        def out_row0(k):
            blk = wid * _NBLK + k
            f = blk // _GPF
            g = blk % _GPF
            return f * (_EMB // 8) * (_B // 128) * 8 + g * 64

        def gather(k, b):
            pltpu.async_copy(
                table_hbm.at[idx_v.at[pl.ds(k * _BLK, _BLK)]],
                gbufs[b],
                gsems[b],
            )

        def wait_gather(k, b):
            pltpu.make_async_copy(
                table_hbm.at[idx_v.at[pl.ds(k * _BLK, _BLK)]],
                gbufs[b],
                gsems[b],
            ).wait()

        def out_dmas(k, wait):
            r0 = out_row0(k)
            for dt in range(4):
                cp = pltpu.make_async_copy(
                    tbuf.at[pl.ds(dt * 64, 64), pl.ds(0, 128)],
                    out_hbm.at[pl.ds(r0 + dt * 1024, 64), :],
                    tsems[dt % 2],
                )
                if wait:
                    cp.wait()
                else:
                    cp.start()

        def shuffle(b):
            gbuf = gbufs[b]

            @plsc.parallel_loop(0, _BLK // 8, step=1, unroll=2)
            def _(i):
                btl = i // 16
                bc0 = (i % 16) * 8
                rv0 = rvec0 + btl * 8
                rv16 = rv0 + 128
                for j in range(8):
                    row = i * 8 + j
                    col = jnp.zeros((16,), jnp.int32) + (bc0 + j)
                    plsc.store_scatter(
                        tbuf, [rv0, col], gbuf[row, pl.ds(0, 16)]
                    )
                    plsc.store_scatter(
                        tbuf, [rv16, col], gbuf[row, pl.ds(16, 16)]
                    )

        pltpu.sync_copy(idx_hbm.at[pl.ds(base, per_w)], idx_v)
        gather(0, 0)
        gather(1, 1)
        for k in range(_NBLK):
            b = k % 2
            wait_gather(k, b)
            if k >= 1:
                out_dmas(k - 1, wait=True)
            shuffle(b)
            if k + 2 < _NBLK:
                gather(k + 2, b)
            out_dmas(k, wait=False)
        out_dmas(_NBLK - 1, wait=True)

    return emb_kernel


def kernel(holder, table):
    b, f = holder.shape
    idx = holder.T.reshape(-1).astype(jnp.int32)
    v = table.shape[0]
    tail128 = lax.slice(table, (v - 64, 0), (v, _EMB)).reshape(16, 128)
    tbl = _make_transpose()(table.T, tail128).reshape(v + 64, _EMB)
    out128 = _make_gather()(idx, tbl)
    out = (
        out128.reshape(f, _EMB // 8, b // 128, 8, 128)
        .transpose(2, 4, 0, 1, 3)
        .reshape(b, f, _EMB)
    )
    return out


# speedup vs baseline: 1.0852x; 1.0852x over previous
import functools

import jax
import jax.numpy as jnp
from jax import lax
from jax.experimental import pallas as pl
from jax.experimental.pallas import tpu as pltpu
from jax.experimental.pallas import tpu_sc as plsc

_EMB = 32
_NUM_CORES = 2
_NUM_SUBCORES = 16
_NW = _NUM_CORES * _NUM_SUBCORES
_BLK = 1024
_B = 16384
_F = 26
_GPF = _B // _BLK
_NBLK = _F * _GPF // _NW


_NTILE = 7812
_TPW = _NTILE // _NW


def _make_transpose():
    mesh = plsc.VectorSubcoreMesh(core_axis_name="c", subcore_axis_name="s")

    @functools.partial(
        pl.kernel,
        mesh=mesh,
        out_type=jax.ShapeDtypeStruct((250016, 128), jnp.float32),
        scratch_types=[
            [pltpu.VMEM((32, 128), jnp.float32) for _ in range(2)],
            [pltpu.VMEM((32, 128), jnp.float32) for _ in range(2)],
            pltpu.VMEM((16, 128), jnp.float32),
            [pltpu.SemaphoreType.DMA for _ in range(2)],
            [pltpu.SemaphoreType.DMA for _ in range(2)],
        ],
        compiler_params=pltpu.CompilerParams(
            use_tc_tiling_on_sc=True, needs_layout_passes=False
        ),
    )
    def trans_kernel(tabt, tail, out, ibufs, tbufs, tailbuf, isems, osems):
        wid = lax.axis_index("s") * _NUM_CORES + lax.axis_index("c")
        base = wid * _TPW
        iota = lax.iota(jnp.int32, 16)

        def issue_in(tt, b):
            pltpu.async_copy(
                tabt.at[:, pl.ds(pl.multiple_of(tt * 128, 128), 128)],
                ibufs[b],
                isems[b],
            )

        def wait_in(tt, b):
            pltpu.make_async_copy(
                tabt.at[:, pl.ds(pl.multiple_of(tt * 128, 128), 128)],
                ibufs[b],
                isems[b],
            ).wait()

        def issue_out(tt, b):
            pltpu.async_copy(
                tbufs[b],
                out.at[pl.ds(pl.multiple_of(tt * 32, 32), 32), :],
                osems[b],
            )

        def wait_out(tt, b):
            pltpu.make_async_copy(
                tbufs[b],
                out.at[pl.ds(pl.multiple_of(tt * 32, 32), 32), :],
                osems[b],
            ).wait()

        def transpose(b):
            @plsc.parallel_loop(0, 32, step=1, unroll=4)
            def _(r):
                c4 = r * 4
                for jj in range(4):
                    colv = jnp.zeros((16,), jnp.int32) + (c4 + jj)
                    for d0 in (0, 16):
                        vals = plsc.load_gather(ibufs[b], [iota + d0, colv])
                        tbufs[b][r, pl.ds(jj * 32 + d0, 16)] = vals

        issue_in(base, 0)
        issue_in(base + 1, 1)

        def body(kk, carry):
            for b in range(2):
                k = kk * 2 + b
                wait_in(base + k, b)

                @pl.when(kk >= 1)
                def _():
                    wait_out(base + k - 2, b)

                transpose(b)

                @pl.when(kk < _TPW // 2 - 1)
                def _():
                    issue_in(base + k + 2, b)

                issue_out(base + k, b)
            return carry

        lax.fori_loop(0, _TPW // 2, body, 0)
        for b in range(2):
            wait_out(base + _TPW - 2 + b, b)

        @pl.when(wid < _NTILE - _TPW * _NW)
        def _():
            tt = _TPW * _NW + wid
            pltpu.async_copy(
                tabt.at[:, pl.ds(pl.multiple_of(tt * 128, 128), 128)],
                ibufs[0],
                isems[0],
            ).wait()
            transpose(0)
            pltpu.async_copy(
                tbufs[0],
                out.at[pl.ds(pl.multiple_of(tt * 32, 32), 32), :],
                osems[0],
            ).wait()

        @pl.when(wid == 4)
        def _():
            pltpu.sync_copy(tail, tailbuf)
            pltpu.sync_copy(tailbuf, out.at[pl.ds(_NTILE * 32, 16), :])

    return trans_kernel


def _make_gather():
    n = _B * _F
    per_w = n // _NW
    mesh = plsc.VectorSubcoreMesh(core_axis_name="c", subcore_axis_name="s")

    @functools.partial(
        pl.kernel,
        mesh=mesh,
        out_type=jax.ShapeDtypeStruct((_F * _EMB * _B // 128, 128), jnp.float32),
        scratch_types=[
            pltpu.VMEM((per_w,), jnp.int32),
            [pltpu.VMEM((_BLK, _EMB), jnp.float32) for _ in range(2)],
            pltpu.VMEM((_BLK * _EMB // 128, 129), jnp.float32),
            [pltpu.SemaphoreType.DMA for _ in range(2)],
            [pltpu.SemaphoreType.DMA for _ in range(2)],
        ],
        compiler_params=pltpu.CompilerParams(
            use_tc_tiling_on_sc=False, needs_layout_passes=False
        ),
    )
    def emb_kernel(idx_hbm, table_hbm, out_hbm, idx_v, gbufs, tbuf, gsems, tsems):
        wid = lax.axis_index("s") * _NUM_CORES + lax.axis_index("c")
        base = wid * per_w
        iota = lax.iota(jnp.int32, 16)
        rvec0 = (iota // 8) * 64 + iota % 8

        def out_row0(k):
            blk = wid * _NBLK + k
            f = blk // _GPF
            g = blk % _GPF
            return f * (_EMB // 8) * (_B // 128) * 8 + g * 64

        def gather(k, b):
            pltpu.async_copy(
                table_hbm.at[idx_v.at[pl.ds(k * _BLK, _BLK)]],
                gbufs[b],
                gsems[b],
            )

        def wait_gather(k, b):
            pltpu.make_async_copy(
                table_hbm.at[idx_v.at[pl.ds(k * _BLK, _BLK)]],
                gbufs[b],
                gsems[b],
            ).wait()

        def out_dmas(k, wait):
            r0 = out_row0(k)
            for dt in range(4):
                cp = pltpu.make_async_copy(
                    tbuf.at[pl.ds(dt * 64, 64), pl.ds(0, 128)],
                    out_hbm.at[pl.ds(r0 + dt * 1024, 64), :],
                    tsems[dt % 2],
                )
                if wait:
                    cp.wait()
                else:
                    cp.start()

        def shuffle(b):
            gbuf = gbufs[b]

            @plsc.parallel_loop(0, _BLK // 8, step=1, unroll=2)
            def _(i):
                btl = i // 16
                bc0 = (i % 16) * 8
                rv0 = rvec0 + btl * 8
                rv16 = rv0 + 128
                for j in range(8):
                    row = i * 8 + j
                    col = jnp.zeros((16,), jnp.int32) + (bc0 + j)
                    plsc.store_scatter(
                        tbuf, [rv0, col], gbuf[row, pl.ds(0, 16)]
                    )
                    plsc.store_scatter(
                        tbuf, [rv16, col], gbuf[row, pl.ds(16, 16)]
                    )

        pltpu.sync_copy(idx_hbm.at[pl.ds(base, per_w)], idx_v)
        gather(0, 0)
        gather(1, 1)
        for k in range(_NBLK):
            b = k % 2
            wait_gather(k, b)
            if k >= 1:
                out_dmas(k - 1, wait=True)
            shuffle(b)
            if k + 2 < _NBLK:
                gather(k + 2, b)
            out_dmas(k, wait=False)
        out_dmas(_NBLK - 1, wait=True)

    return emb_kernel


def kernel(holder, table):
    b, f = holder.shape
    idx = holder.T.reshape(-1).astype(jnp.int32)
    v = table.shape[0]
    tail128 = lax.slice(table, (v - 64, 0), (v, _EMB)).reshape(16, 128)
    tbl = _make_transpose()(table.T, tail128).reshape(v + 64, _EMB)
    out128 = _make_gather()(idx, tbl)
    out = (
        out128.reshape(f, _EMB // 8, b // 128, 8, 128)
        .transpose(2, 4, 0, 1, 3)
        .reshape(b, f, _EMB)
    )
    return out
